# Initial kernel scaffold; baseline (speedup 1.0000x reference)
import jax
import jax.numpy as jnp
from jax.experimental import pallas as pl


def kernel(x, edge_index, W1, b1, W2, b2):
    raise NotImplementedError("write your pallas kernel here")


# speedup vs baseline: 10.4355x; 10.4355x over previous
import functools

import jax
import jax.numpy as jnp
from jax import lax
from jax.experimental import pallas as pl
from jax.experimental.pallas import tpu as pltpu
from jax.experimental.pallas import tpu_sc as plsc

N_NODES = 10000
D = 128
N_EDGES = 320000

NC = 2
NS = 16
NW = NC * NS
CHUNK = 128
CHUNKS_PER_TILE = 79
EDGES_PER_TILE = CHUNK * CHUNKS_PER_TILE
E_PAD = EDGES_PER_TILE * NW
N_PAD = 10240
ROWS_PER_TILE = N_PAD // NS

def _vector_mesh():
    return plsc.VectorSubcoreMesh(core_axis_name="c", subcore_axis_name="s")


def _sc_degree(dst_p, ones128, zeros128):
    @functools.partial(
        pl.kernel,
        out_type=jax.ShapeDtypeStruct((NC * N_PAD, D), jnp.float32),
        mesh=_vector_mesh(),
        scratch_types=[
            pltpu.VMEM((CHUNK,), jnp.int32),
            pltpu.VMEM((CHUNK, D), jnp.float32),
            pltpu.VMEM((CHUNK, D), jnp.float32),
            pltpu.VMEM_SHARED((N_PAD, D), jnp.float32),
        ],
    )
    def deg_kernel(dst_hbm, ones_hbm, zeros_hbm, out_hbm, idx_v, ones_v,
                   zeros_v, acc_sh):
        c = lax.axis_index("c")
        s = lax.axis_index("s")
        wid = c * NS + s
        row0 = s * ROWS_PER_TILE
        pltpu.sync_copy(ones_hbm, ones_v)
        pltpu.sync_copy(zeros_hbm, zeros_v)
        for j in range(ROWS_PER_TILE // CHUNK):
            pltpu.sync_copy(zeros_v, acc_sh.at[pl.ds(row0 + j * CHUNK, CHUNK)])
        plsc.subcore_barrier()
        base = wid * EDGES_PER_TILE

        @pl.loop(0, CHUNKS_PER_TILE)
        def _(k):
            pltpu.sync_copy(dst_hbm.at[pl.ds(base + k * CHUNK, CHUNK)], idx_v)
            pltpu.sync_copy(ones_v, acc_sh.at[idx_v], add=True)

        plsc.subcore_barrier()
        pltpu.sync_copy(
            acc_sh.at[pl.ds(row0, ROWS_PER_TILE)],
            out_hbm.at[pl.ds(c * N_PAD + row0, ROWS_PER_TILE)])

    return deg_kernel(dst_p, ones128, zeros128)


def _sc_edge_pass(g, src_p, dst_p, zeros128):
    @functools.partial(
        pl.kernel,
        out_type=jax.ShapeDtypeStruct((NC * N_PAD, D), jnp.float32),
        mesh=_vector_mesh(),
        scratch_types=[
            pltpu.VMEM((CHUNK,), jnp.int32),
            pltpu.VMEM((CHUNK,), jnp.int32),
            pltpu.VMEM((CHUNK, D), jnp.float32),
            pltpu.VMEM((CHUNK, D), jnp.float32),
            pltpu.VMEM_SHARED((N_PAD, D), jnp.float32),
            pltpu.SemaphoreType.DMA,
        ],
    )
    def edge_kernel(g_hbm, src_hbm, dst_hbm, zeros_hbm, out_hbm, sidx_v,
                    didx_v, rows_v, zeros_v, acc_sh, sem):
        c = lax.axis_index("c")
        s = lax.axis_index("s")
        wid = c * NS + s
        row0 = s * ROWS_PER_TILE
        pltpu.sync_copy(zeros_hbm, zeros_v)
        for j in range(ROWS_PER_TILE // CHUNK):
            pltpu.sync_copy(zeros_v, acc_sh.at[pl.ds(row0 + j * CHUNK, CHUNK)])
        plsc.subcore_barrier()
        base = wid * EDGES_PER_TILE

        @pl.loop(0, CHUNKS_PER_TILE)
        def _(k):
            pltpu.sync_copy(src_hbm.at[pl.ds(base + k * CHUNK, CHUNK)], sidx_v)
            pltpu.sync_copy(dst_hbm.at[pl.ds(base + k * CHUNK, CHUNK)], didx_v)
            pltpu.async_copy(g_hbm.at[sidx_v], rows_v, sem).wait()
            pltpu.sync_copy(rows_v, acc_sh.at[didx_v], add=True)

        plsc.subcore_barrier()
        pltpu.sync_copy(
            acc_sh.at[pl.ds(row0, ROWS_PER_TILE)],
            out_hbm.at[pl.ds(c * N_PAD + row0, ROWS_PER_TILE)])

    return edge_kernel(g, src_p, dst_p, zeros128)


_MM_BLOCK = 1024


def _tc_matmul(x_p, W):
    def mm_kernel(x_ref, w_ref, o_ref):
        o_ref[...] = jnp.dot(x_ref[...], w_ref[...],
                             preferred_element_type=jnp.float32)

    return pl.pallas_call(
        mm_kernel,
        grid=(N_PAD // _MM_BLOCK,),
        in_specs=[
            pl.BlockSpec((_MM_BLOCK, D), lambda i: (i, 0)),
            pl.BlockSpec((D, D), lambda i: (0, 0)),
        ],
        out_specs=pl.BlockSpec((_MM_BLOCK, D), lambda i: (i, 0)),
        out_shape=jax.ShapeDtypeStruct((N_PAD, D), jnp.float32),
    )(x_p, W)


def _dinv_block(d0_ref, d1_ref):
    deg = d0_ref[:, 0:1] + d1_ref[:, 0:1] + 1.0
    return lax.rsqrt(deg)


def _tc_scale(h, dega0, dega1):
    def scale_kernel(h_ref, d0_ref, d1_ref, o_ref):
        o_ref[...] = h_ref[...] * _dinv_block(d0_ref, d1_ref)

    return pl.pallas_call(
        scale_kernel,
        grid=(N_PAD // _MM_BLOCK,),
        in_specs=[
            pl.BlockSpec((_MM_BLOCK, D), lambda i: (i, 0)),
            pl.BlockSpec((_MM_BLOCK, D), lambda i: (i, 0)),
            pl.BlockSpec((_MM_BLOCK, D), lambda i: (i, 0)),
        ],
        out_specs=pl.BlockSpec((_MM_BLOCK, D), lambda i: (i, 0)),
        out_shape=jax.ShapeDtypeStruct((N_PAD, D), jnp.float32),
    )(h, dega0, dega1)


def _tc_combine_mm(acc0, acc1, g1, dega0, dega1, b1, W2):
    def comb_kernel(a0_ref, a1_ref, g_ref, d0_ref, d1_ref, b_ref, w_ref,
                    o_ref):
        dinv = _dinv_block(d0_ref, d1_ref)
        z = dinv * (a0_ref[...] + a1_ref[...] + g_ref[...]) + b_ref[...]
        z = jnp.maximum(z, 0.0)
        h2 = jnp.dot(z, w_ref[...], preferred_element_type=jnp.float32)
        o_ref[...] = h2 * dinv

    return pl.pallas_call(
        comb_kernel,
        grid=(N_PAD // _MM_BLOCK,),
        in_specs=[
            pl.BlockSpec((_MM_BLOCK, D), lambda i: (i, 0)),
            pl.BlockSpec((_MM_BLOCK, D), lambda i: (i, 0)),
            pl.BlockSpec((_MM_BLOCK, D), lambda i: (i, 0)),
            pl.BlockSpec((_MM_BLOCK, D), lambda i: (i, 0)),
            pl.BlockSpec((_MM_BLOCK, D), lambda i: (i, 0)),
            pl.BlockSpec((1, D), lambda i: (0, 0)),
            pl.BlockSpec((D, D), lambda i: (0, 0)),
        ],
        out_specs=pl.BlockSpec((_MM_BLOCK, D), lambda i: (i, 0)),
        out_shape=jax.ShapeDtypeStruct((N_PAD, D), jnp.float32),
    )(acc0, acc1, g1, dega0, dega1, b1, W2)


_FIN_BLOCK = 2000


def _tc_final(acc0, acc1, g2, dega0, dega1, b2):
    def fin_kernel(a0_ref, a1_ref, g_ref, d0_ref, d1_ref, b_ref, o_ref):
        dinv = _dinv_block(d0_ref, d1_ref)
        z = dinv * (a0_ref[...] + a1_ref[...] + g_ref[...]) + b_ref[...]
        o_ref[...] = jnp.maximum(z, 0.0)

    return pl.pallas_call(
        fin_kernel,
        grid=(N_NODES // _FIN_BLOCK,),
        in_specs=[
            pl.BlockSpec((_FIN_BLOCK, D), lambda i: (i, 0)),
            pl.BlockSpec((_FIN_BLOCK, D), lambda i: (i, 0)),
            pl.BlockSpec((_FIN_BLOCK, D), lambda i: (i, 0)),
            pl.BlockSpec((_FIN_BLOCK, D), lambda i: (i, 0)),
            pl.BlockSpec((_FIN_BLOCK, D), lambda i: (i, 0)),
            pl.BlockSpec((1, D), lambda i: (0, 0)),
        ],
        out_specs=pl.BlockSpec((_FIN_BLOCK, D), lambda i: (i, 0)),
        out_shape=jax.ShapeDtypeStruct((N_NODES, D), jnp.float32),
    )(acc0, acc1, g2, dega0, dega1, b2)


def kernel(x, edge_index, W1, b1, W2, b2):
    src = edge_index[0].astype(jnp.int32)
    dst = edge_index[1].astype(jnp.int32)
    pad = jnp.full((E_PAD - N_EDGES,), N_NODES, dtype=jnp.int32)
    src_p = jnp.concatenate([src, pad])
    dst_p = jnp.concatenate([dst, pad])
    x_p = jnp.pad(x, ((0, N_PAD - N_NODES), (0, 0)))
    ones128 = jnp.ones((CHUNK, D), jnp.float32)
    zeros128 = jnp.zeros((CHUNK, D), jnp.float32)
    b1r = b1.reshape(1, D)
    b2r = b2.reshape(1, D)

    dega = _sc_degree(dst_p, ones128, zeros128)
    dega0, dega1 = dega[:N_PAD], dega[N_PAD:]
    h1 = _tc_matmul(x_p, W1)
    g1 = _tc_scale(h1, dega0, dega1)
    acc = _sc_edge_pass(g1, src_p, dst_p, zeros128)
    g2 = _tc_combine_mm(acc[:N_PAD], acc[N_PAD:], g1, dega0, dega1, b1r, W2)
    acc2 = _sc_edge_pass(g2, src_p, dst_p, zeros128)
    out = _tc_final(acc2[:N_PAD], acc2[N_PAD:], g2, dega0, dega1, b2r)
    return out
